# baseline (speedup 1.0000x reference)
import functools

import jax
import jax.numpy as jnp
from jax import lax
from jax.experimental import pallas as pl
from jax.experimental.pallas import tpu as pltpu
from jax.experimental.pallas import tpu_sc as plsc

F32 = jnp.float32

GROUP = 32
C = 96
H = 224
W = 224
PIX = 3136
PIX4 = 784
HW = H * W
KK = 21

RB = 448
LBLK = 3584


def _dot(a, b, dims):
    return jax.lax.dot_general(a, b, (dims, ((), ())),
                               precision=jax.lax.Precision.HIGHEST,
                               preferred_element_type=F32)


def _pool_body(x_ref, xpT_ref, xpreT_ref):
    xb = x_ref[0]
    p = jax.lax.broadcasted_iota(jnp.int32, (W, W // 4), 0)
    wq = jax.lax.broadcasted_iota(jnp.int32, (W, W // 4), 1)
    B = jnp.where((p // 4) == wq, 1.0 / 16.0, 0.0).astype(F32)
    g = jax.lax.broadcasted_iota(jnp.int32, (GROUP, C), 0)
    c = jax.lax.broadcasted_iota(jnp.int32, (GROUP, C), 1)
    M = jnp.where((c // 3) == g, 1.0 / 3.0, 0.0).astype(F32)
    for j in range(2):
        s = jnp.sum(xb[:, 4 * j:4 * j + 4, :], axis=1)
        xpT_blk = _dot(B, s, ((0,), (1,)))
        xpT_ref[56 * j:56 * (j + 1), :] = xpT_blk
        xpreT_ref[56 * j:56 * (j + 1), :] = _dot(xpT_blk, M, ((1,), (1,)))


def _pool(x):
    return pl.pallas_call(
        _pool_body,
        grid=(H // 8,),
        in_specs=[pl.BlockSpec((1, C, 8, W), lambda i: (0, 0, i, 0))],
        out_specs=[
            pl.BlockSpec((112, C), lambda i: (i, 0)),
            pl.BlockSpec((112, GROUP), lambda i: (i, 0)),
        ],
        out_shape=[
            jax.ShapeDtypeStruct((PIX, C), F32),
            jax.ShapeDtypeStruct((PIX, GROUP), F32),
        ],
    )(x)


def _k2_body(kw_ref, kb_ref, xpT_ref, k2_ref):
    k2_ref[...] = _dot(kw_ref[...], xpT_ref[...], ((1,), (0,))) + kb_ref[...]


def _k2(kw, kb2, xpT):
    return pl.pallas_call(
        _k2_body,
        out_shape=jax.ShapeDtypeStruct((PIX4, C), F32),
    )(kw, kb2, xpT)


NLANE = 16


def _topk_sc(adj):
    mesh = plsc.VectorSubcoreMesh(core_axis_name="c", subcore_axis_name="s")

    @functools.partial(
        pl.kernel, mesh=mesh,
        out_type=jax.ShapeDtypeStruct((GROUP, GROUP), F32),
        scratch_types=[pltpu.VMEM((GROUP,), F32),
                       pltpu.VMEM((GROUP,), F32)],
    )
    def body(adj_hbm, out_hbm, row_v, out_v):
        row = lax.axis_index("s") * 2 + lax.axis_index("c")
        pltpu.sync_copy(adj_hbm.at[row], row_v)
        a = row_v[pl.ds(0, NLANE)]
        b = row_v[pl.ds(NLANE, NLANE)]
        ja = lax.broadcasted_iota(jnp.int32, (NLANE,), 0)
        jb = ja + NLANE
        ra = jnp.zeros((NLANE,), jnp.int32)
        rb = jnp.zeros((NLANE,), jnp.int32)
        one = jnp.ones((NLANE,), jnp.int32)
        zero = jnp.zeros((NLANE,), jnp.int32)
        for k in range(GROUP):
            src = a if k < NLANE else b
            sv = jnp.full((NLANE,), src[k % NLANE], F32)
            ra = ra + jnp.where(sv > a, one, zero) \
                    + jnp.where((sv == a) & (ja > k), one, zero)
            rb = rb + jnp.where(sv > b, one, zero) \
                    + jnp.where((sv == b) & (jb > k), one, zero)
        keep_a = ra < KK
        keep_b = rb < KK
        fzero = jnp.zeros((NLANE,), F32)
        ea = jnp.where(keep_a, jnp.exp(a), fzero)
        eb = jnp.where(keep_b, jnp.exp(b), fzero)
        dn = lax.GatherDimensionNumbers(offset_dims=(),
                                        collapsed_slice_dims=(0,),
                                        start_index_map=(0,))
        t = ea + eb
        for sh in (8, 4, 2, 1):
            t = t + lax.gather(t, (ja ^ sh)[:, None], dn, slice_sizes=(1,),
                               mode=lax.GatherScatterMode.PROMISE_IN_BOUNDS)
        out_v[pl.ds(0, NLANE)] = ea / t
        out_v[pl.ds(NLANE, NLANE)] = eb / t
        pltpu.sync_copy(out_v, out_hbm.at[row])

    return body(adj)


def _gcn_body(xpreT_ref, adjf_ref, gvw_ref, gvb_ref, gw_ref, gb_ref,
              gout_ref, acc_ref):
    i = pl.program_id(0)

    @pl.when(i == 0)
    def _():
        acc_ref[...] = jnp.zeros_like(acc_ref)

    value_t = _dot(gvw_ref[...], xpreT_ref[...], ((1,), (0,))) + gvb_ref[...]
    acc_ref[...] += _dot(value_t, gw_ref[...], ((0,), (0,)))

    @pl.when(i == (PIX // RB) - 1)
    def _():
        gout_ref[...] = (_dot(adjf_ref[...], acc_ref[...], ((1,), (0,)))
                         + gb_ref[...])


def _gcn(xpreT, adjf, gvw, gvb2, gw, gb2):
    nsteps = PIX // RB
    return pl.pallas_call(
        _gcn_body,
        grid=(nsteps,),
        in_specs=[
            pl.BlockSpec((PIX, GROUP), lambda i: (0, 0)),
            pl.BlockSpec((GROUP, GROUP), lambda i: (0, 0)),
            pl.BlockSpec((RB, PIX), lambda i: (i, 0)),
            pl.BlockSpec((RB, 1), lambda i: (i, 0)),
            pl.BlockSpec((RB, PIX), lambda i: (i, 0)),
            pl.BlockSpec((1, PIX), lambda i: (0, 0)),
        ],
        out_specs=pl.BlockSpec((GROUP, PIX), lambda i: (0, 0)),
        out_shape=jax.ShapeDtypeStruct((GROUP, PIX), F32),
        scratch_shapes=[pltpu.VMEM((GROUP, PIX), F32)],
    )(xpreT, adjf, gvw, gvb2, gw, gb2)


def _tu_body(gout_ref, k2_ref, qw_ref, qb_ref, vw_ref, vb_ref,
             tu_ref, atten_ref):
    i = pl.program_id(0)

    @pl.when(i == 0)
    def _():
        q2 = _dot(qw_ref[...], gout_ref[...], ((1,), (1,))) + qb_ref[...]
        rawa = _dot(k2_ref[...], q2, ((0,), (0,)))
        atten_ref[...] = jax.nn.softmax(rawa, axis=-1)

    v2t = _dot(gout_ref[...], vw_ref[...], ((1,), (1,))) + vb_ref[0]
    tu_ref[...] = _dot(v2t, atten_ref[...], ((0,), (1,)))


def _tu(gout, k2, qw, qb2, vw, vb2):
    nsteps = PIX // RB
    return pl.pallas_call(
        _tu_body,
        grid=(nsteps,),
        in_specs=[
            pl.BlockSpec((GROUP, PIX), lambda i: (0, 0)),
            pl.BlockSpec((PIX4, C), lambda i: (0, 0)),
            pl.BlockSpec((PIX4, PIX), lambda i: (0, 0)),
            pl.BlockSpec((PIX4, 1), lambda i: (0, 0)),
            pl.BlockSpec((RB, PIX), lambda i: (i, 0)),
            pl.BlockSpec((1, 1, RB), lambda i: (i, 0, 0)),
        ],
        out_specs=pl.BlockSpec((RB, C), lambda i: (i, 0)),
        out_shape=jax.ShapeDtypeStruct((PIX, C), F32),
        scratch_shapes=[pltpu.VMEM((C, GROUP), F32)],
    )(gout, k2, qw, qb2, vw, vb2)


HB = 16


def _out_body(tu2_ref, x_ref, out_ref, g_ref):
    i = pl.program_id(0)

    @pl.when(i == 0)
    def _():
        g_ref[...] = _dot(tu2_ref[...], tu2_ref[...], ((1,), (1,)))

    gmat = g_ref[...]
    for j in range(HB):
        out_ref[0, :, j, :] = _dot(gmat, x_ref[0, :, j, :], ((1,), (0,)))


def _final(tu2, x):
    nsteps = H // HB
    return pl.pallas_call(
        _out_body,
        grid=(nsteps,),
        in_specs=[
            pl.BlockSpec((C, PIX), lambda i: (0, 0)),
            pl.BlockSpec((1, C, HB, W), lambda i: (0, 0, i, 0)),
        ],
        out_specs=pl.BlockSpec((1, C, HB, W), lambda i: (0, 0, i, 0)),
        out_shape=jax.ShapeDtypeStruct((1, C, H, W), F32),
        scratch_shapes=[pltpu.VMEM((C, C), F32)],
    )(tu2, x)


@jax.jit
def kernel(x, gcn_weight, gcn_bias, gcn_qw, gcn_qb, gcn_kw, gcn_kb,
           gcn_vw, gcn_vb, qw, qb, kw, kb, vw, vb):
    xpT, xpreT = _pool(x)

    xp_m = x.reshape(1, C, H // 4, 4, W // 4, 4).mean(axis=(3, 5))
    xpre_m = xp_m.reshape(1, GROUP, C // GROUP, PIX).mean(axis=2)
    xt_m = jnp.transpose(xpre_m, (0, 2, 1))
    q_m = jnp.einsum('oc,bcl->bol', gcn_qw, xt_m) + gcn_qb[None, :, None]
    k_m = jnp.einsum('oc,bcl->bol', gcn_kw, xt_m) + gcn_kb[None, :, None]
    kt_m = jnp.transpose(k_m, (0, 2, 1))
    adj = jax.nn.softmax(jnp.einsum('bsp,bpt->bst', kt_m, q_m), axis=-1)[0]

    adjf = _topk_sc(adj)
    k2 = _k2(kw, kb.reshape(PIX4, 1), xpT)

    gout = _gcn(xpreT, adjf, gcn_vw, gcn_vb.reshape(PIX, 1),
                gcn_weight, gcn_bias.reshape(1, PIX))

    tuT = _tu(gout, k2, qw, qb.reshape(PIX4, 1),
              vw, vb.reshape(PIX // RB, 1, RB))

    tu2 = tuT.reshape(C, PIX)

    return _final(tu2, x)


# speedup vs baseline: 1.1490x; 1.1490x over previous
import functools

import jax
import jax.numpy as jnp
from jax import lax
from jax.experimental import pallas as pl
from jax.experimental.pallas import tpu as pltpu
from jax.experimental.pallas import tpu_sc as plsc

F32 = jnp.float32

GROUP = 32
C = 96
H = 224
W = 224
PIX = 3136
PIX4 = 784
HW = H * W
KK = 21

RB = 448
LBLK = 3584


def _dot(a, b, dims):
    return jax.lax.dot_general(a, b, (dims, ((), ())),
                               precision=jax.lax.Precision.HIGHEST,
                               preferred_element_type=F32)


def _pool_body(x_ref, xpT_ref, xpreT_ref):
    xb = x_ref[0]
    p = jax.lax.broadcasted_iota(jnp.int32, (W, W // 4), 0)
    wq = jax.lax.broadcasted_iota(jnp.int32, (W, W // 4), 1)
    B = jnp.where((p // 4) == wq, 1.0 / 16.0, 0.0).astype(F32)
    g = jax.lax.broadcasted_iota(jnp.int32, (GROUP, C), 0)
    c = jax.lax.broadcasted_iota(jnp.int32, (GROUP, C), 1)
    M = jnp.where((c // 3) == g, 1.0 / 3.0, 0.0).astype(F32)
    for j in range(2):
        s = jnp.sum(xb[:, 4 * j:4 * j + 4, :], axis=1)
        xpT_blk = _dot(B, s, ((0,), (1,)))
        xpT_ref[56 * j:56 * (j + 1), :] = xpT_blk
        xpreT_ref[56 * j:56 * (j + 1), :] = _dot(xpT_blk, M, ((1,), (1,)))


def _pool(x):
    return pl.pallas_call(
        _pool_body,
        grid=(H // 8,),
        in_specs=[pl.BlockSpec((1, C, 8, W), lambda i: (0, 0, i, 0))],
        out_specs=[
            pl.BlockSpec((112, C), lambda i: (i, 0)),
            pl.BlockSpec((112, GROUP), lambda i: (i, 0)),
        ],
        out_shape=[
            jax.ShapeDtypeStruct((PIX, C), F32),
            jax.ShapeDtypeStruct((PIX, GROUP), F32),
        ],
    )(x)


def _k2_body(kw_ref, kb_ref, xpT_ref, k2_ref):
    k2_ref[...] = _dot(kw_ref[...], xpT_ref[...], ((1,), (0,))) + kb_ref[...]


def _k2(kw, kb2, xpT):
    return pl.pallas_call(
        _k2_body,
        out_shape=jax.ShapeDtypeStruct((PIX4, C), F32),
    )(kw, kb2, xpT)


NLANE = 16


def _topk_sc(adj):
    mesh = plsc.VectorSubcoreMesh(core_axis_name="c", subcore_axis_name="s")

    @functools.partial(
        pl.kernel, mesh=mesh,
        out_type=jax.ShapeDtypeStruct((GROUP, GROUP), F32),
        scratch_types=[pltpu.VMEM((GROUP,), F32),
                       pltpu.VMEM((GROUP,), F32)],
    )
    def body(adj_hbm, out_hbm, row_v, out_v):
        row = lax.axis_index("s") * 2 + lax.axis_index("c")
        pltpu.sync_copy(adj_hbm.at[row], row_v)
        a = row_v[pl.ds(0, NLANE)]
        b = row_v[pl.ds(NLANE, NLANE)]
        ja = lax.broadcasted_iota(jnp.int32, (NLANE,), 0)
        jb = ja + NLANE
        ra = jnp.zeros((NLANE,), jnp.int32)
        rb = jnp.zeros((NLANE,), jnp.int32)
        one = jnp.ones((NLANE,), jnp.int32)
        zero = jnp.zeros((NLANE,), jnp.int32)
        for k in range(GROUP):
            src = a if k < NLANE else b
            sv = jnp.full((NLANE,), src[k % NLANE], F32)
            ra = ra + jnp.where(sv > a, one, zero) \
                    + jnp.where((sv == a) & (ja > k), one, zero)
            rb = rb + jnp.where(sv > b, one, zero) \
                    + jnp.where((sv == b) & (jb > k), one, zero)
        keep_a = ra < KK
        keep_b = rb < KK
        fzero = jnp.zeros((NLANE,), F32)
        ea = jnp.where(keep_a, jnp.exp(a), fzero)
        eb = jnp.where(keep_b, jnp.exp(b), fzero)
        dn = lax.GatherDimensionNumbers(offset_dims=(),
                                        collapsed_slice_dims=(0,),
                                        start_index_map=(0,))
        t = ea + eb
        for sh in (8, 4, 2, 1):
            t = t + lax.gather(t, (ja ^ sh)[:, None], dn, slice_sizes=(1,),
                               mode=lax.GatherScatterMode.PROMISE_IN_BOUNDS)
        out_v[pl.ds(0, NLANE)] = ea / t
        out_v[pl.ds(NLANE, NLANE)] = eb / t
        pltpu.sync_copy(out_v, out_hbm.at[row])

    return body(adj)


def _gcn_body(xpreT_ref, adjf_ref, gvw_ref, gvb_ref, gw_ref, gb_ref,
              gout_ref, acc_ref):
    i = pl.program_id(0)

    @pl.when(i == 0)
    def _():
        acc_ref[...] = jnp.zeros_like(acc_ref)

    value_t = _dot(gvw_ref[...], xpreT_ref[...], ((1,), (0,))) + gvb_ref[...]
    acc_ref[...] += _dot(value_t, gw_ref[...], ((0,), (0,)))

    @pl.when(i == (PIX // RB) - 1)
    def _():
        gout_ref[...] = (_dot(adjf_ref[...], acc_ref[...], ((1,), (0,)))
                         + gb_ref[...])


def _gcn(xpreT, adjf, gvw, gvb2, gw, gb2):
    nsteps = PIX // RB
    return pl.pallas_call(
        _gcn_body,
        grid=(nsteps,),
        in_specs=[
            pl.BlockSpec((PIX, GROUP), lambda i: (0, 0)),
            pl.BlockSpec((GROUP, GROUP), lambda i: (0, 0)),
            pl.BlockSpec((RB, PIX), lambda i: (i, 0)),
            pl.BlockSpec((RB, 1), lambda i: (i, 0)),
            pl.BlockSpec((RB, PIX), lambda i: (i, 0)),
            pl.BlockSpec((1, PIX), lambda i: (0, 0)),
        ],
        out_specs=pl.BlockSpec((GROUP, PIX), lambda i: (0, 0)),
        out_shape=jax.ShapeDtypeStruct((GROUP, PIX), F32),
        scratch_shapes=[pltpu.VMEM((GROUP, PIX), F32)],
    )(xpreT, adjf, gvw, gvb2, gw, gb2)


def _tu_body(gout_ref, k2_ref, qw_ref, qb_ref, vw_ref, vb_ref,
             tu_ref, atten_ref):
    i = pl.program_id(0)

    @pl.when(i == 0)
    def _():
        q2 = _dot(qw_ref[...], gout_ref[...], ((1,), (1,))) + qb_ref[...]
        rawa = _dot(k2_ref[...], q2, ((0,), (0,)))
        atten_ref[...] = jax.nn.softmax(rawa, axis=-1)

    v2t = _dot(gout_ref[...], vw_ref[...], ((1,), (1,))) + vb_ref[0]
    tu_ref[...] = _dot(v2t, atten_ref[...], ((0,), (1,)))


def _tu(gout, k2, qw, qb2, vw, vb2):
    nsteps = PIX // RB
    return pl.pallas_call(
        _tu_body,
        grid=(nsteps,),
        in_specs=[
            pl.BlockSpec((GROUP, PIX), lambda i: (0, 0)),
            pl.BlockSpec((PIX4, C), lambda i: (0, 0)),
            pl.BlockSpec((PIX4, PIX), lambda i: (0, 0)),
            pl.BlockSpec((PIX4, 1), lambda i: (0, 0)),
            pl.BlockSpec((RB, PIX), lambda i: (i, 0)),
            pl.BlockSpec((1, 1, RB), lambda i: (i, 0, 0)),
        ],
        out_specs=pl.BlockSpec((RB, C), lambda i: (i, 0)),
        out_shape=jax.ShapeDtypeStruct((PIX, C), F32),
        scratch_shapes=[pltpu.VMEM((C, GROUP), F32)],
    )(gout, k2, qw, qb2, vw, vb2)


HB = 16


def _out_body(tu2_ref, x_ref, out_ref, g_ref):
    i = pl.program_id(0)

    @pl.when(i == 0)
    def _():
        g_ref[...] = _dot(tu2_ref[...], tu2_ref[...], ((1,), (1,)))

    gmat = g_ref[...]
    for j in range(HB):
        out_ref[0, :, j, :] = _dot(gmat, x_ref[0, :, j, :], ((1,), (0,)))


def _final(tu2, x):
    nsteps = H // HB
    return pl.pallas_call(
        _out_body,
        grid=(nsteps,),
        in_specs=[
            pl.BlockSpec((C, PIX), lambda i: (0, 0)),
            pl.BlockSpec((1, C, HB, W), lambda i: (0, 0, i, 0)),
        ],
        out_specs=pl.BlockSpec((1, C, HB, W), lambda i: (0, 0, i, 0)),
        out_shape=jax.ShapeDtypeStruct((1, C, H, W), F32),
        scratch_shapes=[pltpu.VMEM((C, C), F32)],
    )(tu2, x)


@jax.jit
def kernel(x, gcn_weight, gcn_bias, gcn_qw, gcn_qb, gcn_kw, gcn_kb,
           gcn_vw, gcn_vb, qw, qb, kw, kb, vw, vb):
    xpT, xpreT = _pool(x)

    xp_m = lax.reduce_window(x, 0.0, lax.add, (1, 1, 4, 4), (1, 1, 4, 4),
                             'VALID') * (1.0 / 16.0)
    xpre_m = xp_m.reshape(1, GROUP, C // GROUP, PIX).mean(axis=2)
    xt_m = jnp.transpose(xpre_m, (0, 2, 1))
    q_m = jnp.einsum('oc,bcl->bol', gcn_qw, xt_m) + gcn_qb[None, :, None]
    k_m = jnp.einsum('oc,bcl->bol', gcn_kw, xt_m) + gcn_kb[None, :, None]
    kt_m = jnp.transpose(k_m, (0, 2, 1))
    adj = jax.nn.softmax(jnp.einsum('bsp,bpt->bst', kt_m, q_m), axis=-1)[0]

    adjf = _topk_sc(adj)
    k2 = _k2(kw, kb.reshape(PIX4, 1), xpT)

    gout = _gcn(xpreT, adjf, gcn_vw, gcn_vb.reshape(PIX, 1),
                gcn_weight, gcn_bias.reshape(1, PIX))

    tuT = _tu(gout, k2, qw, qb.reshape(PIX4, 1),
              vw, vb.reshape(PIX // RB, 1, RB))

    tu2 = tuT.reshape(C, PIX)

    return _final(tu2, x)


# speedup vs baseline: 1.7681x; 1.5388x over previous
import functools

import jax
import jax.numpy as jnp
from jax import lax
from jax.experimental import pallas as pl
from jax.experimental.pallas import tpu as pltpu
from jax.experimental.pallas import tpu_sc as plsc

F32 = jnp.float32

GROUP = 32
C = 96
H = 224
W = 224
PIX = 3136
PIX4 = 784
HW = H * W
KK = 21

RB = 448
LBLK = 3584


def _dot(a, b, dims, prec=jax.lax.Precision.DEFAULT):
    return jax.lax.dot_general(a, b, (dims, ((), ())),
                               precision=prec,
                               preferred_element_type=F32)


def _pool_body(x_ref, xpT_ref, xpreT_ref):
    xb = x_ref[0]
    p = jax.lax.broadcasted_iota(jnp.int32, (W, W // 4), 0)
    wq = jax.lax.broadcasted_iota(jnp.int32, (W, W // 4), 1)
    B = jnp.where((p // 4) == wq, 1.0 / 16.0, 0.0).astype(F32)
    g = jax.lax.broadcasted_iota(jnp.int32, (GROUP, C), 0)
    c = jax.lax.broadcasted_iota(jnp.int32, (GROUP, C), 1)
    M = jnp.where((c // 3) == g, 1.0 / 3.0, 0.0).astype(F32)
    for j in range(2):
        s = jnp.sum(xb[:, 4 * j:4 * j + 4, :], axis=1)
        xpT_blk = _dot(B, s, ((0,), (1,)))
        xpT_ref[56 * j:56 * (j + 1), :] = xpT_blk
        xpreT_ref[56 * j:56 * (j + 1), :] = _dot(xpT_blk, M, ((1,), (1,)))


def _pool(x):
    return pl.pallas_call(
        _pool_body,
        grid=(H // 8,),
        in_specs=[pl.BlockSpec((1, C, 8, W), lambda i: (0, 0, i, 0))],
        out_specs=[
            pl.BlockSpec((112, C), lambda i: (i, 0)),
            pl.BlockSpec((112, GROUP), lambda i: (i, 0)),
        ],
        out_shape=[
            jax.ShapeDtypeStruct((PIX, C), F32),
            jax.ShapeDtypeStruct((PIX, GROUP), F32),
        ],
    )(x)


def _k2_body(kw_ref, kb_ref, xpT_ref, k2_ref):
    k2_ref[...] = _dot(kw_ref[...], xpT_ref[...], ((1,), (0,))) + kb_ref[...]


def _k2(kw, kb2, xpT):
    return pl.pallas_call(
        _k2_body,
        out_shape=jax.ShapeDtypeStruct((PIX4, C), F32),
    )(kw, kb2, xpT)


NLANE = 16


def _topk_sc(adj):
    mesh = plsc.VectorSubcoreMesh(core_axis_name="c", subcore_axis_name="s")

    @functools.partial(
        pl.kernel, mesh=mesh,
        out_type=jax.ShapeDtypeStruct((GROUP, GROUP), F32),
        scratch_types=[pltpu.VMEM((GROUP,), F32),
                       pltpu.VMEM((GROUP,), F32)],
    )
    def body(adj_hbm, out_hbm, row_v, out_v):
        row = lax.axis_index("s") * 2 + lax.axis_index("c")
        pltpu.sync_copy(adj_hbm.at[row], row_v)
        a = row_v[pl.ds(0, NLANE)]
        b = row_v[pl.ds(NLANE, NLANE)]
        ja = lax.broadcasted_iota(jnp.int32, (NLANE,), 0)
        jb = ja + NLANE
        ra = jnp.zeros((NLANE,), jnp.int32)
        rb = jnp.zeros((NLANE,), jnp.int32)
        one = jnp.ones((NLANE,), jnp.int32)
        zero = jnp.zeros((NLANE,), jnp.int32)
        for k in range(GROUP):
            src = a if k < NLANE else b
            sv = jnp.full((NLANE,), src[k % NLANE], F32)
            ra = ra + jnp.where(sv > a, one, zero) \
                    + jnp.where((sv == a) & (ja > k), one, zero)
            rb = rb + jnp.where(sv > b, one, zero) \
                    + jnp.where((sv == b) & (jb > k), one, zero)
        keep_a = ra < KK
        keep_b = rb < KK
        fzero = jnp.zeros((NLANE,), F32)
        ea = jnp.where(keep_a, jnp.exp(a), fzero)
        eb = jnp.where(keep_b, jnp.exp(b), fzero)
        dn = lax.GatherDimensionNumbers(offset_dims=(),
                                        collapsed_slice_dims=(0,),
                                        start_index_map=(0,))
        t = ea + eb
        for sh in (8, 4, 2, 1):
            t = t + lax.gather(t, (ja ^ sh)[:, None], dn, slice_sizes=(1,),
                               mode=lax.GatherScatterMode.PROMISE_IN_BOUNDS)
        out_v[pl.ds(0, NLANE)] = ea / t
        out_v[pl.ds(NLANE, NLANE)] = eb / t
        pltpu.sync_copy(out_v, out_hbm.at[row])

    return body(adj)


def _gcn_body(xpreT_ref, adjf_ref, gvw_ref, gvb_ref, gw_ref, gb_ref,
              gout_ref, acc_ref):
    i = pl.program_id(0)

    @pl.when(i == 0)
    def _():
        acc_ref[...] = jnp.zeros_like(acc_ref)

    value_t = _dot(gvw_ref[...], xpreT_ref[...], ((1,), (0,))) + gvb_ref[...]
    acc_ref[...] += _dot(value_t, gw_ref[...], ((0,), (0,)))

    @pl.when(i == (PIX // RB) - 1)
    def _():
        gout_ref[...] = (_dot(adjf_ref[...], acc_ref[...], ((1,), (0,)))
                         + gb_ref[...])


def _gcn(xpreT, adjf, gvw, gvb2, gw, gb2):
    nsteps = PIX // RB
    return pl.pallas_call(
        _gcn_body,
        grid=(nsteps,),
        in_specs=[
            pl.BlockSpec((PIX, GROUP), lambda i: (0, 0)),
            pl.BlockSpec((GROUP, GROUP), lambda i: (0, 0)),
            pl.BlockSpec((RB, PIX), lambda i: (i, 0)),
            pl.BlockSpec((RB, 1), lambda i: (i, 0)),
            pl.BlockSpec((RB, PIX), lambda i: (i, 0)),
            pl.BlockSpec((1, PIX), lambda i: (0, 0)),
        ],
        out_specs=pl.BlockSpec((GROUP, PIX), lambda i: (0, 0)),
        out_shape=jax.ShapeDtypeStruct((GROUP, PIX), F32),
        scratch_shapes=[pltpu.VMEM((GROUP, PIX), F32)],
    )(xpreT, adjf, gvw, gvb2, gw, gb2)


def _tu_body(gout_ref, k2_ref, qw_ref, qb_ref, vw_ref, vb_ref,
             tu_ref, atten_ref):
    i = pl.program_id(0)

    @pl.when(i == 0)
    def _():
        q2 = _dot(qw_ref[...], gout_ref[...], ((1,), (1,))) + qb_ref[...]
        rawa = _dot(k2_ref[...], q2, ((0,), (0,)))
        atten_ref[...] = jax.nn.softmax(rawa, axis=-1)

    v2t = _dot(gout_ref[...], vw_ref[...], ((1,), (1,))) + vb_ref[0]
    tu_ref[...] = _dot(v2t, atten_ref[...], ((0,), (1,)))


def _tu(gout, k2, qw, qb2, vw, vb2):
    nsteps = PIX // RB
    return pl.pallas_call(
        _tu_body,
        grid=(nsteps,),
        in_specs=[
            pl.BlockSpec((GROUP, PIX), lambda i: (0, 0)),
            pl.BlockSpec((PIX4, C), lambda i: (0, 0)),
            pl.BlockSpec((PIX4, PIX), lambda i: (0, 0)),
            pl.BlockSpec((PIX4, 1), lambda i: (0, 0)),
            pl.BlockSpec((RB, PIX), lambda i: (i, 0)),
            pl.BlockSpec((1, 1, RB), lambda i: (i, 0, 0)),
        ],
        out_specs=pl.BlockSpec((RB, C), lambda i: (i, 0)),
        out_shape=jax.ShapeDtypeStruct((PIX, C), F32),
        scratch_shapes=[pltpu.VMEM((C, GROUP), F32)],
    )(gout, k2, qw, qb2, vw, vb2)


HB = 16


def _out_body(tu2_ref, x_ref, out_ref, g_ref):
    i = pl.program_id(0)

    @pl.when(i == 0)
    def _():
        g_ref[...] = _dot(tu2_ref[...], tu2_ref[...], ((1,), (1,)),
                          prec=jax.lax.Precision.HIGHEST)

    gmat = g_ref[...]
    for j in range(HB):
        out_ref[0, :, j, :] = _dot(gmat, x_ref[0, :, j, :], ((1,), (0,)))


def _final(tu2, x):
    nsteps = H // HB
    return pl.pallas_call(
        _out_body,
        grid=(nsteps,),
        in_specs=[
            pl.BlockSpec((C, PIX), lambda i: (0, 0)),
            pl.BlockSpec((1, C, HB, W), lambda i: (0, 0, i, 0)),
        ],
        out_specs=pl.BlockSpec((1, C, HB, W), lambda i: (0, 0, i, 0)),
        out_shape=jax.ShapeDtypeStruct((1, C, H, W), F32),
        scratch_shapes=[pltpu.VMEM((C, C), F32)],
    )(tu2, x)


@jax.jit
def kernel(x, gcn_weight, gcn_bias, gcn_qw, gcn_qb, gcn_kw, gcn_kb,
           gcn_vw, gcn_vb, qw, qb, kw, kb, vw, vb):
    xpT, xpreT = _pool(x)

    xp_m = lax.reduce_window(x, 0.0, lax.add, (1, 1, 4, 4), (1, 1, 4, 4),
                             'VALID') * (1.0 / 16.0)
    xpre_m = xp_m.reshape(1, GROUP, C // GROUP, PIX).mean(axis=2)
    xt_m = jnp.transpose(xpre_m, (0, 2, 1))
    q_m = jnp.einsum('oc,bcl->bol', gcn_qw, xt_m) + gcn_qb[None, :, None]
    k_m = jnp.einsum('oc,bcl->bol', gcn_kw, xt_m) + gcn_kb[None, :, None]
    kt_m = jnp.transpose(k_m, (0, 2, 1))
    adj = jax.nn.softmax(jnp.einsum('bsp,bpt->bst', kt_m, q_m), axis=-1)[0]

    adjf = _topk_sc(adj)
    k2 = _k2(kw, kb.reshape(PIX4, 1), xpT)

    gout = _gcn(xpreT, adjf, gcn_vw, gcn_vb.reshape(PIX, 1),
                gcn_weight, gcn_bias.reshape(1, PIX))

    tuT = _tu(gout, k2, qw, qb.reshape(PIX4, 1),
              vw, vb.reshape(PIX // RB, 1, RB))

    tu2 = tuT.reshape(C, PIX)

    return _final(tu2, x)
